# baseline (speedup 1.0000x reference)
import functools

import jax
import jax.numpy as jnp
from jax import lax
from jax.experimental import pallas as pl
from jax.experimental.pallas import tpu as pltpu
from jax.experimental.pallas import tpu_sc as plsc

DVEC = 64
SENT = 50
XP = 64
NC, NS, LANES = 2, 16, 16
NW = NC * NS


def _table_body(emb_ref, learn_ref, fixed_ref, cooc_ref, s_ref, x_ref):
    x_ref[...] = jnp.zeros_like(x_ref)
    x = jax.nn.softmax(emb_ref[...], axis=-1) * learn_ref[...] + fixed_ref[...]
    x_ref[pl.ds(0, SENT), :] = x
    xp = x_ref[pl.ds(0, XP), :]
    for o in range(3):
        t = lax.dot_general(xp, cooc_ref[o], (((1,), (0,)), ((), ())),
                            preferred_element_type=jnp.float32)
        s_ref[o] = lax.dot_general(t, x_ref[...], (((1,), (1,)), ((), ())),
                                   preferred_element_type=jnp.float32)


def _score_table(emb_rows, learn_rows, fixed_rows, cooc):
    return pl.pallas_call(
        _table_body,
        in_specs=[
            pl.BlockSpec((SENT, DVEC), lambda: (0, 0)),
            pl.BlockSpec((SENT, 1), lambda: (0, 0)),
            pl.BlockSpec((SENT, DVEC), lambda: (0, 0)),
            pl.BlockSpec((3, DVEC, DVEC), lambda: (0, 0, 0)),
        ],
        out_specs=pl.BlockSpec((3, XP, 128), lambda: (0, 0, 0)),
        out_shape=jax.ShapeDtypeStruct((3, XP, 128), jnp.float32),
        scratch_shapes=[pltpu.VMEM((128, DVEC), jnp.float32)],
    )(emb_rows, learn_rows[:, None], fixed_rows, cooc)


def _make_sc_kernel(k, n1):
    per_w = k // NW
    ch = per_w // LANES
    mesh = plsc.VectorSubcoreMesh(core_axis_name="c", subcore_axis_name="s")

    @functools.partial(
        pl.kernel,
        out_type=jax.ShapeDtypeStruct((k,), jnp.float32),
        mesh=mesh,
        compiler_params=pltpu.CompilerParams(needs_layout_passes=False),
        scratch_types=[
            pltpu.VMEM((3 * XP * 128,), jnp.float32),
            pltpu.VMEM((per_w * n1,), jnp.int32),
            pltpu.VMEM((4 * SENT * LANES,), jnp.int32),
            pltpu.VMEM((per_w,), jnp.float32),
        ],
    )
    def sc_kernel(s_hbm, code_hbm, out_hbm, s_v, code_v, ptr_v, sc_v):
        w = lax.axis_index("s") * NC + lax.axis_index("c")
        pltpu.sync_copy(s_hbm, s_v)
        pltpu.sync_copy(code_hbm.at[pl.ds(w * per_w * n1, per_w * n1)], code_v)
        lanes = lax.iota(jnp.int32, LANES)
        lanes_n1 = lanes * n1
        sl = SENT * LANES
        NI = 4
        for j in range(0, ch, NI):
            for s in range(SENT):
                splat = jnp.full((LANES,), s, jnp.int32)
                for q in range(NI):
                    plsc.store_scatter(ptr_v, [q * sl + s * LANES + lanes], splat)

            def step(i, accs, j=j):
                out = []
                for q in range(NI):
                    c = plsc.load_gather(code_v, [((j + q) * LANES * n1 + i) + lanes_n1])
                    av = c & 63
                    fv = (c >> 6) & 63
                    op = c >> 12
                    pf = plsc.load_gather(ptr_v, [q * sl + fv * LANES + lanes])
                    pa = plsc.load_gather(ptr_v, [q * sl + av * LANES + lanes])
                    v = plsc.load_gather(s_v, [(op << 13) + (pf << 7) + pa])
                    plsc.store_scatter(ptr_v, [q * sl + fv * LANES + lanes], pa,
                                       mask=op == 2)
                    out.append(accs[q] + v)
                return tuple(out)

            zero = jnp.zeros((LANES,), jnp.float32)
            accs = lax.fori_loop(0, n1, step, (zero,) * NI)
            for q in range(NI):
                plsc.store_scatter(sc_v, [(j + q) * LANES + lanes], accs[q])
        pltpu.sync_copy(sc_v, out_hbm.at[pl.ds(w * per_w, per_w)])

    return sc_kernel


def kernel(emb_weight, learn_vectors, fixed_vectors, cooc, ids, ops, ix_func, ix_arg):
    onehot = (ids[:, None] == jnp.arange(emb_weight.shape[0])[None, :]).astype(jnp.float32)
    hi = jax.lax.Precision.HIGH
    emb_rows = jnp.matmul(onehot, emb_weight, precision=hi)
    fixed_rows = jnp.matmul(onehot, fixed_vectors, precision=hi)
    learn_rows = jnp.take(learn_vectors, ids, axis=0)
    s_pad = _score_table(emb_rows, learn_rows, fixed_rows, cooc)
    k, n1 = ops.shape
    code = (ops << 12) | (ix_func << 6) | ix_arg
    sc_fn = _make_sc_kernel(k, n1)
    return sc_fn(s_pad.reshape(3 * XP * 128), code.reshape(-1))
